# baseline (speedup 1.0000x reference)
import functools

import jax
import jax.numpy as jnp
from jax import lax
from jax.experimental import pallas as pl
from jax.experimental.pallas import tpu as pltpu
from jax.experimental.pallas import tpu_sc as plsc

N = 10000
N_PAD = 10240
E = 160000
E_PAD = 163840
F32 = jnp.float32
DH = 128


def _zero_rows(rows_ref, nrows, ncol16):
    z = jnp.zeros((16,), F32)

    def body(r, t):
        for k in range(ncol16):
            rows_ref[r, pl.ds(k * 16, 16)] = z
        return t

    lax.fori_loop(0, nrows, body, 0)


def _agg_pipeline(x_hbm, sidx, didx, buf0, buf1, acc, semg, sems, n_chunks):
    n2 = n_chunks // 2
    pltpu.async_copy(x_hbm.at[sidx.at[0]], buf0, semg)

    def body(jj, t):
        j0 = 2 * jj
        pltpu.make_async_copy(x_hbm.at[sidx.at[j0]], buf0, semg).wait()
        pltpu.async_copy(buf0, acc.at[didx.at[j0]], sems, add=True)

        @pl.when(jj > 0)
        def _():
            pltpu.make_async_copy(buf1, acc.at[didx.at[j0]], sems).wait()
        pltpu.async_copy(x_hbm.at[sidx.at[j0 + 1]], buf1, semg)

        pltpu.make_async_copy(x_hbm.at[sidx.at[j0 + 1]], buf1, semg).wait()
        pltpu.async_copy(buf1, acc.at[didx.at[j0 + 1]], sems, add=True)

        pltpu.make_async_copy(buf0, acc.at[didx.at[j0]], sems).wait()
        jn = lax.select(jj + 1 < n2, j0 + 2, 0)
        pltpu.async_copy(x_hbm.at[sidx.at[jn]], buf0, semg)
        return t

    lax.fori_loop(0, n2, body, 0)
    pltpu.make_async_copy(x_hbm.at[sidx.at[0]], buf0, semg).wait()
    pltpu.make_async_copy(buf1, acc.at[didx.at[0]], sems).wait()


@functools.lru_cache(maxsize=None)
def _make_sc_agg():
    n_chunks = E_PAD // 16 // 128
    phase = n_chunks // 2
    rows_per_tile = N_PAD // 16

    mesh = plsc.VectorSubcoreMesh(core_axis_name="c", subcore_axis_name="s")

    @functools.partial(
        pl.kernel,
        mesh=mesh,
        out_type=[
            jax.ShapeDtypeStruct((N_PAD, DH), F32),
            jax.ShapeDtypeStruct((N_PAD, DH), F32),
        ],
        scratch_types=[
            pltpu.VMEM((phase, 128), jnp.int32),
            pltpu.VMEM((phase, 128), jnp.int32),
            pltpu.VMEM((128, DH), F32),
            pltpu.VMEM((128, DH), F32),
            pltpu.VMEM_SHARED((N_PAD, DH), F32),
            pltpu.SemaphoreType.DMA,
            pltpu.SemaphoreType.DMA,
        ],
    )
    def agg(x0_hbm, x1_hbm, src_hbm, dst_hbm, out0, out1,
            sidx, didx, buf0, buf1, acc, semg, sems):
        c = lax.axis_index("c")
        s = lax.axis_index("s")

        _zero_rows(buf0, 128, DH // 16)
        for kk in range(rows_per_tile // 128):
            pltpu.sync_copy(buf0, acc.at[pl.ds(s * rows_per_tile + kk * 128, 128)])
        plsc.subcore_barrier()

        def chain(x_hbm):
            for ph in range(2):
                pltpu.sync_copy(src_hbm.at[s, pl.ds(ph * phase, phase)], sidx)
                pltpu.sync_copy(dst_hbm.at[s, pl.ds(ph * phase, phase)], didx)
                _agg_pipeline(x_hbm, sidx, didx, buf0, buf1, acc, semg, sems, phase)

        @pl.when(c == 0)
        def _():
            chain(x0_hbm)

        @pl.when(c == 1)
        def _():
            chain(x1_hbm)

        plsc.subcore_barrier()

        r0 = s * rows_per_tile

        @pl.when(c == 0)
        def _():
            pltpu.sync_copy(acc.at[pl.ds(r0, rows_per_tile)],
                            out0.at[pl.ds(r0, rows_per_tile)])

        @pl.when(c == 1)
        def _():
            pltpu.sync_copy(acc.at[pl.ds(r0, rows_per_tile)],
                            out1.at[pl.ds(r0, rows_per_tile)])

    return agg


@functools.lru_cache(maxsize=None)
def _make_sc_topo():
    n_chunks = E_PAD // 32 // 128
    rows_per_tile = N_PAD // 16

    mesh = plsc.VectorSubcoreMesh(core_axis_name="c", subcore_axis_name="s")

    @functools.partial(
        pl.kernel,
        mesh=mesh,
        out_type=[
            jax.ShapeDtypeStruct((N_PAD, DH), F32),
            jax.ShapeDtypeStruct((N_PAD, DH), F32),
        ],
        scratch_types=[
            pltpu.VMEM((n_chunks, 128), jnp.int32),
            pltpu.VMEM((n_chunks, 128), jnp.int32),
            pltpu.VMEM((128, DH), F32),
            pltpu.VMEM((128, DH), F32),
            pltpu.VMEM_SHARED((N_PAD, DH), F32),
            pltpu.SemaphoreType.DMA,
            pltpu.SemaphoreType.DMA,
        ],
    )
    def topo_agg(x_hbm, src_hbm, dst_hbm, out0, out1,
                 sidx, didx, buf0, buf1, acc, semg, sems):
        c = lax.axis_index("c")
        s = lax.axis_index("s")
        wid = s * 2 + c

        _zero_rows(buf0, 128, DH // 16)
        for kk in range(rows_per_tile // 128):
            pltpu.sync_copy(buf0, acc.at[pl.ds(s * rows_per_tile + kk * 128, 128)])
        plsc.subcore_barrier()

        pltpu.sync_copy(src_hbm.at[wid], sidx)
        pltpu.sync_copy(dst_hbm.at[wid], didx)

        _agg_pipeline(x_hbm, sidx, didx, buf0, buf1, acc, semg, sems, n_chunks)

        plsc.subcore_barrier()

        r0 = s * rows_per_tile

        @pl.when(c == 0)
        def _():
            pltpu.sync_copy(acc.at[pl.ds(r0, rows_per_tile)],
                            out0.at[pl.ds(r0, rows_per_tile)])

        @pl.when(c == 1)
        def _():
            pltpu.sync_copy(acc.at[pl.ds(r0, rows_per_tile)],
                            out1.at[pl.ds(r0, rows_per_tile)])

    return topo_agg


def _full(shape):
    return pl.BlockSpec(shape, lambda i: tuple(0 for _ in shape))


def _rows(shape_tail, bs):
    return pl.BlockSpec((bs,) + shape_tail, lambda i: (i,) + tuple(0 for _ in shape_tail))


def _mm(a, b):
    return jnp.dot(a, b, preferred_element_type=F32)


def _tc_stage_a(geom, pts, topo, t0, t1,
                Wg1, bg1, Wg2, bg2, Wb1, bb1, Wb2, bb2, Wt, bt):
    R = 80
    grid = (N // R,)

    def body(geom_r, pts_r, topo_r, t0_r, t1_r,
             Wg1_r, bg1_r, Wg2_r, bg2_r, Wb1_r, bb1_r, Wb2_r, bb2_r,
             Wt_r, bt_r, o0_r, o1_r, inv_r):
        g = jax.nn.relu(_mm(geom_r[...], Wg1_r[...]) + bg1_r[...])
        geom_out = jax.nn.relu(_mm(g, Wg2_r[...]) + bg2_r[...])

        p = pts_r[...].reshape(R * 64, 3)
        p = jax.nn.relu(_mm(p, Wb1_r[...]) + bb1_r[...])
        p = jax.nn.relu(_mm(p, Wb2_r[...]) + bb2_r[...])
        pm = jnp.max(p.reshape(R, 64, 64), axis=1)

        asum = t0_r[...] + t1_r[...]
        deg = asum[:, 1:2]
        inv = 1.0 / jnp.maximum(deg, 1.0)
        tagg = asum[:, 0:1] * inv
        topo_out = jax.nn.relu((topo_r[...] + tagg) * Wt_r[...] + bt_r[...])

        x0 = jnp.concatenate([geom_out, pm, topo_out], axis=1)
        z32 = jnp.zeros((R, 32), F32)
        o0_r[...] = jnp.concatenate([x0[:, :96], z32], axis=1)
        o1_r[...] = jnp.concatenate([x0[:, 96:], z32], axis=1)
        inv_r[...] = inv

    return pl.pallas_call(
        body,
        grid=grid,
        in_specs=[
            _rows((13,), R), _rows((64, 3), R), _rows((1,), R),
            _rows((DH,), R), _rows((DH,), R),
            _full((13, 32)), _full((1, 32)), _full((32, 64)), _full((1, 64)),
            _full((3, 64)), _full((1, 64)), _full((64, 64)), _full((1, 64)),
            _full((1, 64)), _full((1, 64)),
        ],
        out_specs=[_rows((DH,), R), _rows((DH,), R), _rows((1,), R)],
        out_shape=[
            jax.ShapeDtypeStruct((N_PAD, DH), F32),
            jax.ShapeDtypeStruct((N_PAD, DH), F32),
            jax.ShapeDtypeStruct((N_PAD, 1), F32),
        ],
    )(geom, pts, topo, t0, t1, Wg1, bg1, Wg2, bg2, Wb1, bb1, Wb2, bb2, Wt, bt)


def _tc_gcn(x0, x1, a0, a1, inv, W, b, din):
    R = 256
    dh_in = din // 2
    grid = (N_PAD // R,)

    def body(x0_r, x1_r, a0_r, a1_r, inv_r, W_r, b_r, o0_r, o1_r):
        x = jnp.concatenate([x0_r[...][:, :dh_in], x1_r[...][:, :dh_in]], axis=1)
        a = jnp.concatenate([a0_r[...][:, :dh_in], a1_r[...][:, :dh_in]], axis=1) * inv_r[...]
        y = jax.nn.relu(_mm(x + a, W_r[...]) + b_r[...])
        o0_r[...] = y[:, :128]
        o1_r[...] = y[:, 128:]

    return pl.pallas_call(
        body,
        grid=grid,
        in_specs=[
            _rows((DH,), R), _rows((DH,), R),
            _rows((DH,), R), _rows((DH,), R),
            _rows((1,), R),
            _full((din, 256)), _full((1, 256)),
        ],
        out_specs=[_rows((DH,), R), _rows((DH,), R)],
        out_shape=[
            jax.ShapeDtypeStruct((N_PAD, DH), F32),
            jax.ShapeDtypeStruct((N_PAD, DH), F32),
        ],
    )(x0, x1, a0, a1, inv, W, b)


def _tc_final(x0, x1, a0, a1, inv, W3, b3, Wf1, bf1, Wf2, bf2, Wf3, bf3):
    R = 80
    grid = (N // R,)

    def body(x0_r, x1_r, a0_r, a1_r, inv_r,
             W3_r, b3_r, Wf1_r, bf1_r, Wf2_r, bf2_r, Wf3_r, bf3_r, o_r):
        x = jnp.concatenate([x0_r[...], x1_r[...]], axis=1)
        a = jnp.concatenate([a0_r[...], a1_r[...]], axis=1) * inv_r[...]
        x3 = jax.nn.relu(_mm(x + a, W3_r[...]) + b3_r[...])
        h = jax.nn.relu(_mm(x3, Wf1_r[...]) + bf1_r[...])
        h = jax.nn.relu(_mm(h, Wf2_r[...]) + bf2_r[...])
        logits = _mm(h, Wf3_r[...]) + bf3_r[...]
        o_r[...] = jax.nn.sigmoid(logits)

    return pl.pallas_call(
        body,
        grid=grid,
        in_specs=[
            _rows((DH,), R), _rows((DH,), R),
            _rows((DH,), R), _rows((DH,), R),
            _rows((1,), R),
            _full((256, 128)), _full((1, 128)),
            _full((128, 128)), _full((1, 128)),
            _full((128, 64)), _full((1, 64)),
            _full((64, 25)), _full((1, 25)),
        ],
        out_specs=[_rows((25,), R)],
        out_shape=[jax.ShapeDtypeStruct((N, 25), F32)],
    )(x0, x1, a0, a1, inv, W3, b3, Wf1, bf1, Wf2, bf2, Wf3, bf3)[0]


def kernel(geom_feat, boundary_pts, topo_feat, face_edge_index,
           Wg1, bg1, Wg2, bg2, Wb1, bb1, Wb2, bb2, Wt, bt,
           W1, b1, W2, b2, W3, b3, Wf1, bf1, Wf2, bf2, Wf3, bf3):
    fe = face_edge_index.astype(jnp.int32)
    pad = jnp.full((E_PAD - E,), N, jnp.int32)
    src = jnp.concatenate([fe[0], pad])
    dst = jnp.concatenate([fe[1], pad])
    src_big = src.reshape(16, E_PAD // 16 // 128, 128)
    dst_big = dst.reshape(16, E_PAD // 16 // 128, 128)
    src_t = src.reshape(32, E_PAD // 32 // 128, 128)
    dst_t = dst.reshape(32, E_PAD // 32 // 128, 128)

    topo128 = jnp.zeros((N_PAD, DH), F32)
    topo128 = topo128.at[:, 1].set(1.0)
    topo128 = topo128.at[:N, 0].set(topo_feat[:, 0])

    t0, t1 = _make_sc_topo()(topo128, src_t, dst_t)

    x0, x1, inv = _tc_stage_a(
        geom_feat, boundary_pts, topo_feat, t0, t1,
        Wg1, bg1.reshape(1, -1), Wg2, bg2.reshape(1, -1),
        Wb1, bb1.reshape(1, -1), Wb2, bb2.reshape(1, -1),
        Wt, bt.reshape(1, -1))

    a0, a1 = _make_sc_agg()(x0, x1, src_big, dst_big)
    y0, y1 = _tc_gcn(x0, x1, a0, a1, inv, W1, b1.reshape(1, -1), 192)

    a0, a1 = _make_sc_agg()(y0, y1, src_big, dst_big)
    z0, z1 = _tc_gcn(y0, y1, a0, a1, inv, W2, b2.reshape(1, -1), 256)

    a0, a1 = _make_sc_agg()(z0, z1, src_big, dst_big)
    out = _tc_final(z0, z1, a0, a1, inv,
                    W3, b3.reshape(1, -1),
                    Wf1[:128], bf1.reshape(1, -1),
                    Wf2, bf2.reshape(1, -1),
                    Wf3, bf3.reshape(1, -1))
    return out


# speedup vs baseline: 1.0710x; 1.0710x over previous
import functools

import jax
import jax.numpy as jnp
from jax import lax
from jax.experimental import pallas as pl
from jax.experimental.pallas import tpu as pltpu
from jax.experimental.pallas import tpu_sc as plsc

N = 10000
N_PAD = 10240
E = 160000
E_PAD = 163840
F32 = jnp.float32
DH = 128


def _zero_rows(rows_ref, nrows, ncol16):
    z = jnp.zeros((16,), F32)

    def body(r, t):
        for k in range(ncol16):
            rows_ref[r, pl.ds(k * 16, 16)] = z
        return t

    lax.fori_loop(0, nrows, body, 0)


def _agg_pipeline(x_hbm, sidx, didx, buf0, buf1, acc, semg, sems, n_chunks):
    n2 = n_chunks // 2
    pltpu.async_copy(x_hbm.at[sidx.at[0]], buf0, semg)

    def body(jj, t):
        j0 = 2 * jj
        pltpu.make_async_copy(x_hbm.at[sidx.at[j0]], buf0, semg).wait()
        pltpu.async_copy(buf0, acc.at[didx.at[j0]], sems, add=True)

        @pl.when(jj > 0)
        def _():
            pltpu.make_async_copy(buf1, acc.at[didx.at[j0]], sems).wait()
        pltpu.async_copy(x_hbm.at[sidx.at[j0 + 1]], buf1, semg)

        pltpu.make_async_copy(x_hbm.at[sidx.at[j0 + 1]], buf1, semg).wait()
        pltpu.async_copy(buf1, acc.at[didx.at[j0 + 1]], sems, add=True)

        pltpu.make_async_copy(buf0, acc.at[didx.at[j0]], sems).wait()
        jn = lax.select(jj + 1 < n2, j0 + 2, 0)
        pltpu.async_copy(x_hbm.at[sidx.at[jn]], buf0, semg)
        return t

    lax.fori_loop(0, n2, body, 0)
    pltpu.make_async_copy(x_hbm.at[sidx.at[0]], buf0, semg).wait()
    pltpu.make_async_copy(buf1, acc.at[didx.at[0]], sems).wait()


@functools.lru_cache(maxsize=None)
def _make_sc_agg():
    n_chunks = E_PAD // 16 // 128
    phase = n_chunks // 2
    rows_per_tile = N_PAD // 16

    mesh = plsc.VectorSubcoreMesh(core_axis_name="c", subcore_axis_name="s")

    @functools.partial(
        pl.kernel,
        mesh=mesh,
        out_type=[
            jax.ShapeDtypeStruct((N_PAD, DH), F32),
            jax.ShapeDtypeStruct((N_PAD, DH), F32),
        ],
        scratch_types=[
            pltpu.VMEM((phase, 128), jnp.int32),
            pltpu.VMEM((phase, 128), jnp.int32),
            pltpu.VMEM((128, DH), F32),
            pltpu.VMEM((128, DH), F32),
            pltpu.VMEM_SHARED((N_PAD, DH), F32),
            pltpu.SemaphoreType.DMA,
            pltpu.SemaphoreType.DMA,
        ],
    )
    def agg(x0_hbm, x1_hbm, src_hbm, dst_hbm, out0, out1,
            sidx, didx, buf0, buf1, acc, semg, sems):
        c = lax.axis_index("c")
        s = lax.axis_index("s")

        _zero_rows(buf0, 128, DH // 16)
        for kk in range(rows_per_tile // 128):
            pltpu.sync_copy(buf0, acc.at[pl.ds(s * rows_per_tile + kk * 128, 128)])
        plsc.subcore_barrier()

        def chain(x_hbm):
            for ph in range(2):
                pltpu.sync_copy(src_hbm.at[s, pl.ds(ph * phase, phase)], sidx)
                pltpu.sync_copy(dst_hbm.at[s, pl.ds(ph * phase, phase)], didx)
                _agg_pipeline(x_hbm, sidx, didx, buf0, buf1, acc, semg, sems, phase)

        @pl.when(c == 0)
        def _():
            chain(x0_hbm)

        @pl.when(c == 1)
        def _():
            chain(x1_hbm)

        plsc.subcore_barrier()

        r0 = s * rows_per_tile

        @pl.when(c == 0)
        def _():
            pltpu.sync_copy(acc.at[pl.ds(r0, rows_per_tile)],
                            out0.at[pl.ds(r0, rows_per_tile)])

        @pl.when(c == 1)
        def _():
            pltpu.sync_copy(acc.at[pl.ds(r0, rows_per_tile)],
                            out1.at[pl.ds(r0, rows_per_tile)])

    return agg


@functools.lru_cache(maxsize=None)
def _make_sc_topo():
    n_chunks = E_PAD // 32 // 128
    rows_per_tile = N_PAD // 16

    mesh = plsc.VectorSubcoreMesh(core_axis_name="c", subcore_axis_name="s")

    @functools.partial(
        pl.kernel,
        mesh=mesh,
        out_type=[
            jax.ShapeDtypeStruct((N_PAD, DH), F32),
            jax.ShapeDtypeStruct((N_PAD, DH), F32),
        ],
        scratch_types=[
            pltpu.VMEM((n_chunks, 128), jnp.int32),
            pltpu.VMEM((n_chunks, 128), jnp.int32),
            pltpu.VMEM((128, DH), F32),
            pltpu.VMEM((128, DH), F32),
            pltpu.VMEM_SHARED((N_PAD, DH), F32),
            pltpu.SemaphoreType.DMA,
            pltpu.SemaphoreType.DMA,
        ],
    )
    def topo_agg(x_hbm, src_hbm, dst_hbm, out0, out1,
                 sidx, didx, buf0, buf1, acc, semg, sems):
        c = lax.axis_index("c")
        s = lax.axis_index("s")
        wid = s * 2 + c

        _zero_rows(buf0, 128, DH // 16)
        for kk in range(rows_per_tile // 128):
            pltpu.sync_copy(buf0, acc.at[pl.ds(s * rows_per_tile + kk * 128, 128)])
        plsc.subcore_barrier()

        pltpu.sync_copy(src_hbm.at[wid], sidx)
        pltpu.sync_copy(dst_hbm.at[wid], didx)

        _agg_pipeline(x_hbm, sidx, didx, buf0, buf1, acc, semg, sems, n_chunks)

        plsc.subcore_barrier()

        r0 = s * rows_per_tile

        @pl.when(c == 0)
        def _():
            pltpu.sync_copy(acc.at[pl.ds(r0, rows_per_tile)],
                            out0.at[pl.ds(r0, rows_per_tile)])

        @pl.when(c == 1)
        def _():
            pltpu.sync_copy(acc.at[pl.ds(r0, rows_per_tile)],
                            out1.at[pl.ds(r0, rows_per_tile)])

    return topo_agg


def _full(shape):
    return pl.BlockSpec(shape, lambda i: tuple(0 for _ in shape))


def _rows(shape_tail, bs):
    return pl.BlockSpec((bs,) + shape_tail, lambda i: (i,) + tuple(0 for _ in shape_tail))


def _mm(a, b):
    return jnp.dot(a, b, preferred_element_type=F32)


def _tc_stage_a1(geom, pts, Wg1, bg1, Wg2, bg2, Wb1, bb1, Wb2, bb2):
    R = 80
    grid = (N // R,)

    def body(geom_r, pts_r,
             Wg1_r, bg1_r, Wg2_r, bg2_r, Wb1_r, bb1_r, Wb2_r, bb2_r, o0_r):
        g = jax.nn.relu(_mm(geom_r[...], Wg1_r[...]) + bg1_r[...])
        geom_out = jax.nn.relu(_mm(g, Wg2_r[...]) + bg2_r[...])

        p = pts_r[...].reshape(R * 64, 3)
        p = jax.nn.relu(_mm(p, Wb1_r[...]) + bb1_r[...])
        p = jax.nn.relu(_mm(p, Wb2_r[...]) + bb2_r[...])
        pm = jnp.max(p.reshape(R, 64, 64), axis=1)

        o0_r[...] = jnp.concatenate([geom_out, pm], axis=1)

    return pl.pallas_call(
        body,
        grid=grid,
        in_specs=[
            _rows((13,), R), _rows((64, 3), R),
            _full((13, 32)), _full((1, 32)), _full((32, 64)), _full((1, 64)),
            _full((3, 64)), _full((1, 64)), _full((64, 64)), _full((1, 64)),
        ],
        out_specs=[_rows((DH,), R)],
        out_shape=[jax.ShapeDtypeStruct((N_PAD, DH), F32)],
    )(geom, pts, Wg1, bg1, Wg2, bg2, Wb1, bb1, Wb2, bb2)[0]


def _tc_stage_a2(topo, t0, t1, Wt, bt):
    R = 80
    grid = (N // R,)

    def body(topo_r, t0_r, t1_r, Wt_r, bt_r, o1_r, inv_r):
        asum = t0_r[...] + t1_r[...]
        deg = asum[:, 1:2]
        inv = 1.0 / jnp.maximum(deg, 1.0)
        tagg = asum[:, 0:1] * inv
        topo_out = jax.nn.relu((topo_r[...] + tagg) * Wt_r[...] + bt_r[...])
        o1_r[...] = jnp.concatenate([topo_out, jnp.zeros((R, 64), F32)], axis=1)
        inv_r[...] = inv

    return pl.pallas_call(
        body,
        grid=grid,
        in_specs=[
            _rows((1,), R), _rows((DH,), R), _rows((DH,), R),
            _full((1, 64)), _full((1, 64)),
        ],
        out_specs=[_rows((DH,), R), _rows((1,), R)],
        out_shape=[
            jax.ShapeDtypeStruct((N_PAD, DH), F32),
            jax.ShapeDtypeStruct((N_PAD, 1), F32),
        ],
    )(topo, t0, t1, Wt, bt)


def _tc_gcn(x0, x1, a0, a1, inv, W, b, w0, w1):
    R = 256
    grid = (N_PAD // R,)

    def body(x0_r, x1_r, a0_r, a1_r, inv_r, W_r, b_r, o0_r, o1_r):
        x = jnp.concatenate([x0_r[...][:, :w0], x1_r[...][:, :w1]], axis=1)
        a = jnp.concatenate([a0_r[...][:, :w0], a1_r[...][:, :w1]], axis=1) * inv_r[...]
        y = jax.nn.relu(_mm(x + a, W_r[...]) + b_r[...])
        o0_r[...] = y[:, :128]
        o1_r[...] = y[:, 128:]

    return pl.pallas_call(
        body,
        grid=grid,
        in_specs=[
            _rows((DH,), R), _rows((DH,), R),
            _rows((DH,), R), _rows((DH,), R),
            _rows((1,), R),
            _full((w0 + w1, 256)), _full((1, 256)),
        ],
        out_specs=[_rows((DH,), R), _rows((DH,), R)],
        out_shape=[
            jax.ShapeDtypeStruct((N_PAD, DH), F32),
            jax.ShapeDtypeStruct((N_PAD, DH), F32),
        ],
    )(x0, x1, a0, a1, inv, W, b)


def _tc_final(x0, x1, a0, a1, inv, W3, b3, Wf1, bf1, Wf2, bf2, Wf3, bf3):
    R = 80
    grid = (N // R,)

    def body(x0_r, x1_r, a0_r, a1_r, inv_r,
             W3_r, b3_r, Wf1_r, bf1_r, Wf2_r, bf2_r, Wf3_r, bf3_r, o_r):
        x = jnp.concatenate([x0_r[...], x1_r[...]], axis=1)
        a = jnp.concatenate([a0_r[...], a1_r[...]], axis=1) * inv_r[...]
        x3 = jax.nn.relu(_mm(x + a, W3_r[...]) + b3_r[...])
        h = jax.nn.relu(_mm(x3, Wf1_r[...]) + bf1_r[...])
        h = jax.nn.relu(_mm(h, Wf2_r[...]) + bf2_r[...])
        logits = _mm(h, Wf3_r[...]) + bf3_r[...]
        o_r[...] = jax.nn.sigmoid(logits)

    return pl.pallas_call(
        body,
        grid=grid,
        in_specs=[
            _rows((DH,), R), _rows((DH,), R),
            _rows((DH,), R), _rows((DH,), R),
            _rows((1,), R),
            _full((256, 128)), _full((1, 128)),
            _full((128, 128)), _full((1, 128)),
            _full((128, 64)), _full((1, 64)),
            _full((64, 25)), _full((1, 25)),
        ],
        out_specs=[_rows((25,), R)],
        out_shape=[jax.ShapeDtypeStruct((N, 25), F32)],
    )(x0, x1, a0, a1, inv, W3, b3, Wf1, bf1, Wf2, bf2, Wf3, bf3)[0]


def kernel(geom_feat, boundary_pts, topo_feat, face_edge_index,
           Wg1, bg1, Wg2, bg2, Wb1, bb1, Wb2, bb2, Wt, bt,
           W1, b1, W2, b2, W3, b3, Wf1, bf1, Wf2, bf2, Wf3, bf3):
    fe = face_edge_index.astype(jnp.int32)
    pad = jnp.full((E_PAD - E,), N, jnp.int32)
    src = jnp.concatenate([fe[0], pad])
    dst = jnp.concatenate([fe[1], pad])
    src_big = src.reshape(16, E_PAD // 16 // 128, 128)
    dst_big = dst.reshape(16, E_PAD // 16 // 128, 128)
    src_t = src.reshape(32, E_PAD // 32 // 128, 128)
    dst_t = dst.reshape(32, E_PAD // 32 // 128, 128)

    topo128 = jnp.zeros((N_PAD, DH), F32)
    topo128 = topo128.at[:, 1].set(1.0)
    topo128 = topo128.at[:N, 0].set(topo_feat[:, 0])

    t0, t1 = _make_sc_topo()(topo128, src_t, dst_t)

    x0 = _tc_stage_a1(
        geom_feat, boundary_pts,
        Wg1, bg1.reshape(1, -1), Wg2, bg2.reshape(1, -1),
        Wb1, bb1.reshape(1, -1), Wb2, bb2.reshape(1, -1))
    x1, inv = _tc_stage_a2(topo_feat, t0, t1, Wt, bt.reshape(1, -1))

    a0, a1 = _make_sc_agg()(x0, x1, src_big, dst_big)
    y0, y1 = _tc_gcn(x0, x1, a0, a1, inv, W1, b1.reshape(1, -1), 128, 64)

    a0, a1 = _make_sc_agg()(y0, y1, src_big, dst_big)
    z0, z1 = _tc_gcn(y0, y1, a0, a1, inv, W2, b2.reshape(1, -1), 128, 128)

    a0, a1 = _make_sc_agg()(z0, z1, src_big, dst_big)
    out = _tc_final(z0, z1, a0, a1, inv,
                    W3, b3.reshape(1, -1),
                    Wf1[:128], bf1.reshape(1, -1),
                    Wf2, bf2.reshape(1, -1),
                    Wf3, bf3.reshape(1, -1))
    return out


# speedup vs baseline: 1.1386x; 1.0632x over previous
import functools

import jax
import jax.numpy as jnp
from jax import lax
from jax.experimental import pallas as pl
from jax.experimental.pallas import tpu as pltpu
from jax.experimental.pallas import tpu_sc as plsc

N = 10000
N_PAD = 10240
E = 160000
E_PAD = 163840
F32 = jnp.float32
DH = 128


def _zero_rows(rows_ref, nrows, ncol16):
    z = jnp.zeros((16,), F32)

    def body(r, t):
        for k in range(ncol16):
            rows_ref[r, pl.ds(k * 16, 16)] = z
        return t

    lax.fori_loop(0, nrows, body, 0)


def _agg_pipeline(x_hbm, sidx, didx, buf0, buf1, acc, semg, sems, n_chunks):
    n2 = n_chunks // 2
    pltpu.async_copy(x_hbm.at[sidx.at[0]], buf0, semg)

    def body(jj, t):
        j0 = 2 * jj
        pltpu.make_async_copy(x_hbm.at[sidx.at[j0]], buf0, semg).wait()
        pltpu.async_copy(buf0, acc.at[didx.at[j0]], sems, add=True)

        @pl.when(jj > 0)
        def _():
            pltpu.make_async_copy(buf1, acc.at[didx.at[j0]], sems).wait()
        pltpu.async_copy(x_hbm.at[sidx.at[j0 + 1]], buf1, semg)

        pltpu.make_async_copy(x_hbm.at[sidx.at[j0 + 1]], buf1, semg).wait()
        pltpu.async_copy(buf1, acc.at[didx.at[j0 + 1]], sems, add=True)

        pltpu.make_async_copy(buf0, acc.at[didx.at[j0]], sems).wait()
        jn = lax.select(jj + 1 < n2, j0 + 2, 0)
        pltpu.async_copy(x_hbm.at[sidx.at[jn]], buf0, semg)
        return t

    lax.fori_loop(0, n2, body, 0)
    pltpu.make_async_copy(x_hbm.at[sidx.at[0]], buf0, semg).wait()
    pltpu.make_async_copy(buf1, acc.at[didx.at[0]], sems).wait()


@functools.lru_cache(maxsize=None)
def _make_sc_agg():
    n_chunks = E_PAD // 16 // 128
    phase = n_chunks // 2
    rows_per_tile = N_PAD // 16

    mesh = plsc.VectorSubcoreMesh(core_axis_name="c", subcore_axis_name="s")

    @functools.partial(
        pl.kernel,
        mesh=mesh,
        out_type=[
            jax.ShapeDtypeStruct((N_PAD, DH), F32),
            jax.ShapeDtypeStruct((N_PAD, DH), F32),
        ],
        scratch_types=[
            pltpu.VMEM((phase, 128), jnp.int32),
            pltpu.VMEM((phase, 128), jnp.int32),
            pltpu.VMEM((128, DH), F32),
            pltpu.VMEM((128, DH), F32),
            pltpu.VMEM_SHARED((N_PAD, DH), F32),
            pltpu.SemaphoreType.DMA,
            pltpu.SemaphoreType.DMA,
        ],
    )
    def agg(x0_hbm, x1_hbm, src_hbm, dst_hbm, out0, out1,
            sidx, didx, buf0, buf1, acc, semg, sems):
        c = lax.axis_index("c")
        s = lax.axis_index("s")

        _zero_rows(buf0, 128, DH // 16)
        for kk in range(rows_per_tile // 128):
            pltpu.sync_copy(buf0, acc.at[pl.ds(s * rows_per_tile + kk * 128, 128)])
        plsc.subcore_barrier()

        def chain(x_hbm):
            for ph in range(2):
                pltpu.sync_copy(src_hbm.at[s, pl.ds(ph * phase, phase)], sidx)
                pltpu.sync_copy(dst_hbm.at[s, pl.ds(ph * phase, phase)], didx)
                _agg_pipeline(x_hbm, sidx, didx, buf0, buf1, acc, semg, sems, phase)

        @pl.when(c == 0)
        def _():
            chain(x0_hbm)

        @pl.when(c == 1)
        def _():
            chain(x1_hbm)

        plsc.subcore_barrier()

        r0 = s * rows_per_tile

        @pl.when(c == 0)
        def _():
            pltpu.sync_copy(acc.at[pl.ds(r0, rows_per_tile)],
                            out0.at[pl.ds(r0, rows_per_tile)])

        @pl.when(c == 1)
        def _():
            pltpu.sync_copy(acc.at[pl.ds(r0, rows_per_tile)],
                            out1.at[pl.ds(r0, rows_per_tile)])

    return agg


@functools.lru_cache(maxsize=None)
def _make_sc_topo():
    n_chunks = E_PAD // 32 // 128
    rows_per_tile = N_PAD // 16

    mesh = plsc.VectorSubcoreMesh(core_axis_name="c", subcore_axis_name="s")

    @functools.partial(
        pl.kernel,
        mesh=mesh,
        out_type=[
            jax.ShapeDtypeStruct((N_PAD, DH), F32),
            jax.ShapeDtypeStruct((N_PAD, DH), F32),
        ],
        scratch_types=[
            pltpu.VMEM((n_chunks, 128), jnp.int32),
            pltpu.VMEM((n_chunks, 128), jnp.int32),
            pltpu.VMEM((128, DH), F32),
            pltpu.VMEM((128, DH), F32),
            pltpu.VMEM_SHARED((N_PAD, DH), F32),
            pltpu.SemaphoreType.DMA,
            pltpu.SemaphoreType.DMA,
        ],
    )
    def topo_agg(x_hbm, src_hbm, dst_hbm, out0, out1,
                 sidx, didx, buf0, buf1, acc, semg, sems):
        c = lax.axis_index("c")
        s = lax.axis_index("s")
        wid = s * 2 + c

        _zero_rows(buf0, 128, DH // 16)
        for kk in range(rows_per_tile // 128):
            pltpu.sync_copy(buf0, acc.at[pl.ds(s * rows_per_tile + kk * 128, 128)])
        plsc.subcore_barrier()

        pltpu.sync_copy(src_hbm.at[wid], sidx)
        pltpu.sync_copy(dst_hbm.at[wid], didx)

        _agg_pipeline(x_hbm, sidx, didx, buf0, buf1, acc, semg, sems, n_chunks)

        plsc.subcore_barrier()

        r0 = s * rows_per_tile

        @pl.when(c == 0)
        def _():
            pltpu.sync_copy(acc.at[pl.ds(r0, rows_per_tile)],
                            out0.at[pl.ds(r0, rows_per_tile)])

        @pl.when(c == 1)
        def _():
            pltpu.sync_copy(acc.at[pl.ds(r0, rows_per_tile)],
                            out1.at[pl.ds(r0, rows_per_tile)])

    return topo_agg


def _full(shape):
    return pl.BlockSpec(shape, lambda i: tuple(0 for _ in shape))


def _rows(shape_tail, bs):
    return pl.BlockSpec((bs,) + shape_tail, lambda i: (i,) + tuple(0 for _ in shape_tail))


def _mm(a, b):
    return jnp.dot(a, b, preferred_element_type=F32)


def _tc_stage_a1(geom, pts, Wg1, bg1, Wg2, bg2, Wb1, bb1, Wb2, bb2):
    R = 200
    grid = (N // R,)

    def body(geom_r, pts_r,
             Wg1_r, bg1_r, Wg2_r, bg2_r, Wb1_r, bb1_r, Wb2_r, bb2_r, o0_r):
        g = jax.nn.relu(_mm(geom_r[...], Wg1_r[...]) + bg1_r[...])
        geom_out = jax.nn.relu(_mm(g, Wg2_r[...]) + bg2_r[...])

        p = pts_r[...].reshape(R * 64, 3)
        p = jax.nn.relu(_mm(p, Wb1_r[...]) + bb1_r[...])
        p = jax.nn.relu(_mm(p, Wb2_r[...]) + bb2_r[...])
        pm = jnp.max(p.reshape(R, 64, 64), axis=1)

        o0_r[...] = jnp.concatenate([geom_out, pm], axis=1)

    return pl.pallas_call(
        body,
        grid=grid,
        in_specs=[
            _rows((13,), R), _rows((64, 3), R),
            _full((13, 32)), _full((1, 32)), _full((32, 64)), _full((1, 64)),
            _full((3, 64)), _full((1, 64)), _full((64, 64)), _full((1, 64)),
        ],
        out_specs=[_rows((DH,), R)],
        out_shape=[jax.ShapeDtypeStruct((N_PAD, DH), F32)],
    )(geom, pts, Wg1, bg1, Wg2, bg2, Wb1, bb1, Wb2, bb2)[0]


def _tc_stage_a2(topo, t0, t1, Wt, bt):
    R = 80
    grid = (N // R,)

    def body(topo_r, t0_r, t1_r, Wt_r, bt_r, o1_r, inv_r):
        asum = t0_r[...] + t1_r[...]
        deg = asum[:, 1:2]
        inv = 1.0 / jnp.maximum(deg, 1.0)
        tagg = asum[:, 0:1] * inv
        topo_out = jax.nn.relu((topo_r[...] + tagg) * Wt_r[...] + bt_r[...])
        o1_r[...] = jnp.concatenate([topo_out, jnp.zeros((R, 64), F32)], axis=1)
        inv_r[...] = inv

    return pl.pallas_call(
        body,
        grid=grid,
        in_specs=[
            _rows((1,), R), _rows((DH,), R), _rows((DH,), R),
            _full((1, 64)), _full((1, 64)),
        ],
        out_specs=[_rows((DH,), R), _rows((1,), R)],
        out_shape=[
            jax.ShapeDtypeStruct((N_PAD, DH), F32),
            jax.ShapeDtypeStruct((N_PAD, 1), F32),
        ],
    )(topo, t0, t1, Wt, bt)


def _tc_gcn(x0, x1, a0, a1, inv, W, b, w0, w1):
    R = 256
    grid = (N_PAD // R,)

    def body(x0_r, x1_r, a0_r, a1_r, inv_r, W_r, b_r, o0_r, o1_r):
        x = jnp.concatenate([x0_r[...][:, :w0], x1_r[...][:, :w1]], axis=1)
        a = jnp.concatenate([a0_r[...][:, :w0], a1_r[...][:, :w1]], axis=1) * inv_r[...]
        y = jax.nn.relu(_mm(x + a, W_r[...]) + b_r[...])
        o0_r[...] = y[:, :128]
        o1_r[...] = y[:, 128:]

    return pl.pallas_call(
        body,
        grid=grid,
        in_specs=[
            _rows((DH,), R), _rows((DH,), R),
            _rows((DH,), R), _rows((DH,), R),
            _rows((1,), R),
            _full((w0 + w1, 256)), _full((1, 256)),
        ],
        out_specs=[_rows((DH,), R), _rows((DH,), R)],
        out_shape=[
            jax.ShapeDtypeStruct((N_PAD, DH), F32),
            jax.ShapeDtypeStruct((N_PAD, DH), F32),
        ],
    )(x0, x1, a0, a1, inv, W, b)


def _tc_final(x0, x1, a0, a1, inv, W3, b3, Wf1, bf1, Wf2, bf2, Wf3, bf3):
    R = 200
    grid = (N // R,)

    def body(x0_r, x1_r, a0_r, a1_r, inv_r,
             W3_r, b3_r, Wf1_r, bf1_r, Wf2_r, bf2_r, Wf3_r, bf3_r, o_r):
        x = jnp.concatenate([x0_r[...], x1_r[...]], axis=1)
        a = jnp.concatenate([a0_r[...], a1_r[...]], axis=1) * inv_r[...]
        x3 = jax.nn.relu(_mm(x + a, W3_r[...]) + b3_r[...])
        h = jax.nn.relu(_mm(x3, Wf1_r[...]) + bf1_r[...])
        h = jax.nn.relu(_mm(h, Wf2_r[...]) + bf2_r[...])
        logits = _mm(h, Wf3_r[...]) + bf3_r[...]
        o_r[...] = jax.nn.sigmoid(logits)

    return pl.pallas_call(
        body,
        grid=grid,
        in_specs=[
            _rows((DH,), R), _rows((DH,), R),
            _rows((DH,), R), _rows((DH,), R),
            _rows((1,), R),
            _full((256, 128)), _full((1, 128)),
            _full((128, 128)), _full((1, 128)),
            _full((128, 64)), _full((1, 64)),
            _full((64, 25)), _full((1, 25)),
        ],
        out_specs=[_rows((25,), R)],
        out_shape=[jax.ShapeDtypeStruct((N, 25), F32)],
    )(x0, x1, a0, a1, inv, W3, b3, Wf1, bf1, Wf2, bf2, Wf3, bf3)[0]


def kernel(geom_feat, boundary_pts, topo_feat, face_edge_index,
           Wg1, bg1, Wg2, bg2, Wb1, bb1, Wb2, bb2, Wt, bt,
           W1, b1, W2, b2, W3, b3, Wf1, bf1, Wf2, bf2, Wf3, bf3):
    fe = face_edge_index.astype(jnp.int32)
    pad = jnp.full((E_PAD - E,), N, jnp.int32)
    src = jnp.concatenate([fe[0], pad])
    dst = jnp.concatenate([fe[1], pad])
    src_big = src.reshape(16, E_PAD // 16 // 128, 128)
    dst_big = dst.reshape(16, E_PAD // 16 // 128, 128)
    src_t = src.reshape(32, E_PAD // 32 // 128, 128)
    dst_t = dst.reshape(32, E_PAD // 32 // 128, 128)

    topo128 = jnp.zeros((N_PAD, DH), F32)
    topo128 = topo128.at[:, 1].set(1.0)
    topo128 = topo128.at[:N, 0].set(topo_feat[:, 0])

    t0, t1 = _make_sc_topo()(topo128, src_t, dst_t)

    x0 = _tc_stage_a1(
        geom_feat, boundary_pts,
        Wg1, bg1.reshape(1, -1), Wg2, bg2.reshape(1, -1),
        Wb1, bb1.reshape(1, -1), Wb2, bb2.reshape(1, -1))
    x1, inv = _tc_stage_a2(topo_feat, t0, t1, Wt, bt.reshape(1, -1))

    a0, a1 = _make_sc_agg()(x0, x1, src_big, dst_big)
    y0, y1 = _tc_gcn(x0, x1, a0, a1, inv, W1, b1.reshape(1, -1), 128, 64)

    a0, a1 = _make_sc_agg()(y0, y1, src_big, dst_big)
    z0, z1 = _tc_gcn(y0, y1, a0, a1, inv, W2, b2.reshape(1, -1), 128, 128)

    a0, a1 = _make_sc_agg()(z0, z1, src_big, dst_big)
    out = _tc_final(z0, z1, a0, a1, inv,
                    W3, b3.reshape(1, -1),
                    Wf1[:128], bf1.reshape(1, -1),
                    Wf2, bf2.reshape(1, -1),
                    Wf3, bf3.reshape(1, -1))
    return out


# speedup vs baseline: 1.1683x; 1.0260x over previous
import functools

import jax
import jax.numpy as jnp
from jax import lax
from jax.experimental import pallas as pl
from jax.experimental.pallas import tpu as pltpu
from jax.experimental.pallas import tpu_sc as plsc

N = 10000
N_PAD = 10240
E = 160000
E_PAD = 163840
F32 = jnp.float32
DH = 128


def _zero_rows(rows_ref, nrows, ncol16):
    z = jnp.zeros((16,), F32)

    def body(r, t):
        for k in range(ncol16):
            rows_ref[r, pl.ds(k * 16, 16)] = z
        return t

    lax.fori_loop(0, nrows, body, 0)


def _agg_pipeline(x_hbm, sidx, didx, buf0, buf1, acc, semg, sems, n_chunks):
    n2 = n_chunks // 2
    pltpu.async_copy(x_hbm.at[sidx.at[0]], buf0, semg)

    def body(jj, t):
        j0 = 2 * jj
        pltpu.make_async_copy(x_hbm.at[sidx.at[j0]], buf0, semg).wait()
        pltpu.async_copy(buf0, acc.at[didx.at[j0]], sems, add=True)

        @pl.when(jj > 0)
        def _():
            pltpu.make_async_copy(buf1, acc.at[didx.at[j0]], sems).wait()
        pltpu.async_copy(x_hbm.at[sidx.at[j0 + 1]], buf1, semg)

        pltpu.make_async_copy(x_hbm.at[sidx.at[j0 + 1]], buf1, semg).wait()
        pltpu.async_copy(buf1, acc.at[didx.at[j0 + 1]], sems, add=True)

        pltpu.make_async_copy(buf0, acc.at[didx.at[j0]], sems).wait()
        jn = lax.select(jj + 1 < n2, j0 + 2, 0)
        pltpu.async_copy(x_hbm.at[sidx.at[jn]], buf0, semg)
        return t

    lax.fori_loop(0, n2, body, 0)
    pltpu.make_async_copy(x_hbm.at[sidx.at[0]], buf0, semg).wait()
    pltpu.make_async_copy(buf1, acc.at[didx.at[0]], sems).wait()


@functools.lru_cache(maxsize=None)
def _make_sc_agg():
    n_chunks = E_PAD // 16 // 128
    phase = n_chunks // 2
    rows_per_tile = N_PAD // 16

    mesh = plsc.VectorSubcoreMesh(core_axis_name="c", subcore_axis_name="s")

    @functools.partial(
        pl.kernel,
        mesh=mesh,
        out_type=[
            jax.ShapeDtypeStruct((N_PAD, DH), F32),
            jax.ShapeDtypeStruct((N_PAD, DH), F32),
        ],
        scratch_types=[
            pltpu.VMEM((phase, 128), jnp.int32),
            pltpu.VMEM((phase, 128), jnp.int32),
            pltpu.VMEM((128, DH), F32),
            pltpu.VMEM((128, DH), F32),
            pltpu.VMEM_SHARED((N_PAD, DH), F32),
            pltpu.SemaphoreType.DMA,
            pltpu.SemaphoreType.DMA,
        ],
    )
    def agg(x0_hbm, x1_hbm, src_hbm, dst_hbm, out0, out1,
            sidx, didx, buf0, buf1, acc, semg, sems):
        c = lax.axis_index("c")
        s = lax.axis_index("s")

        _zero_rows(buf0, 128, DH // 16)
        for kk in range(rows_per_tile // 128):
            pltpu.sync_copy(buf0, acc.at[pl.ds(s * rows_per_tile + kk * 128, 128)])
        plsc.subcore_barrier()

        def chain(x_hbm):
            for ph in range(2):
                pltpu.sync_copy(src_hbm.at[s, pl.ds(ph * phase, phase)], sidx)
                pltpu.sync_copy(dst_hbm.at[s, pl.ds(ph * phase, phase)], didx)
                _agg_pipeline(x_hbm, sidx, didx, buf0, buf1, acc, semg, sems, phase)

        @pl.when(c == 0)
        def _():
            chain(x0_hbm)

        @pl.when(c == 1)
        def _():
            chain(x1_hbm)

        plsc.subcore_barrier()

        r0 = s * rows_per_tile

        @pl.when(c == 0)
        def _():
            pltpu.sync_copy(acc.at[pl.ds(r0, rows_per_tile)],
                            out0.at[pl.ds(r0, rows_per_tile)])

        @pl.when(c == 1)
        def _():
            pltpu.sync_copy(acc.at[pl.ds(r0, rows_per_tile)],
                            out1.at[pl.ds(r0, rows_per_tile)])

    return agg


@functools.lru_cache(maxsize=None)
def _make_sc_topo():
    n_chunks = E_PAD // 32 // 128
    rows_per_tile = N_PAD // 16

    mesh = plsc.VectorSubcoreMesh(core_axis_name="c", subcore_axis_name="s")

    @functools.partial(
        pl.kernel,
        mesh=mesh,
        out_type=[
            jax.ShapeDtypeStruct((N_PAD, DH), F32),
            jax.ShapeDtypeStruct((N_PAD, DH), F32),
        ],
        scratch_types=[
            pltpu.VMEM((n_chunks, 128), jnp.int32),
            pltpu.VMEM((n_chunks, 128), jnp.int32),
            pltpu.VMEM((128, DH), F32),
            pltpu.VMEM((128, DH), F32),
            pltpu.VMEM_SHARED((N_PAD, DH), F32),
            pltpu.SemaphoreType.DMA,
            pltpu.SemaphoreType.DMA,
        ],
    )
    def topo_agg(x_hbm, src_hbm, dst_hbm, out0, out1,
                 sidx, didx, buf0, buf1, acc, semg, sems):
        c = lax.axis_index("c")
        s = lax.axis_index("s")
        wid = s * 2 + c

        _zero_rows(buf0, 128, DH // 16)
        for kk in range(rows_per_tile // 128):
            pltpu.sync_copy(buf0, acc.at[pl.ds(s * rows_per_tile + kk * 128, 128)])
        plsc.subcore_barrier()

        pltpu.sync_copy(src_hbm.at[wid], sidx)
        pltpu.sync_copy(dst_hbm.at[wid], didx)

        _agg_pipeline(x_hbm, sidx, didx, buf0, buf1, acc, semg, sems, n_chunks)

        plsc.subcore_barrier()

        r0 = s * rows_per_tile

        @pl.when(c == 0)
        def _():
            pltpu.sync_copy(acc.at[pl.ds(r0, rows_per_tile)],
                            out0.at[pl.ds(r0, rows_per_tile)])

        @pl.when(c == 1)
        def _():
            pltpu.sync_copy(acc.at[pl.ds(r0, rows_per_tile)],
                            out1.at[pl.ds(r0, rows_per_tile)])

    return topo_agg


def _full(shape):
    return pl.BlockSpec(shape, lambda i: tuple(0 for _ in shape))


def _rows(shape_tail, bs):
    return pl.BlockSpec((bs,) + shape_tail, lambda i: (i,) + tuple(0 for _ in shape_tail))


def _mm(a, b):
    return jnp.dot(a, b, preferred_element_type=F32)


def _tc_stage_a1(geom, pts, Wg1, bg1, Wg2, bg2, Wb1, bb1, Wb2, bb2):
    R = 400
    grid = (N // R,)

    def body(geom_r, pts_r,
             Wg1_r, bg1_r, Wg2_r, bg2_r, Wb1_r, bb1_r, Wb2_r, bb2_r, o0_r):
        g = jax.nn.relu(_mm(geom_r[...], Wg1_r[...]) + bg1_r[...])
        geom_out = jax.nn.relu(_mm(g, Wg2_r[...]) + bg2_r[...])

        p = pts_r[...].reshape(R * 64, 3)
        p = jax.nn.relu(_mm(p, Wb1_r[...]) + bb1_r[...])
        p = jax.nn.relu(_mm(p, Wb2_r[...]) + bb2_r[...])
        pm = jnp.max(p.reshape(R, 64, 64), axis=1)

        o0_r[...] = jnp.concatenate([geom_out, pm], axis=1)

    return pl.pallas_call(
        body,
        grid=grid,
        in_specs=[
            _rows((13,), R), _rows((64, 3), R),
            _full((13, 32)), _full((1, 32)), _full((32, 64)), _full((1, 64)),
            _full((3, 64)), _full((1, 64)), _full((64, 64)), _full((1, 64)),
        ],
        out_specs=[_rows((DH,), R)],
        out_shape=[jax.ShapeDtypeStruct((N_PAD, DH), F32)],
    )(geom, pts, Wg1, bg1, Wg2, bg2, Wb1, bb1, Wb2, bb2)[0]


def _tc_stage_a2(topo, t0, t1, Wt, bt):
    R = 80
    grid = (N // R,)

    def body(topo_r, t0_r, t1_r, Wt_r, bt_r, o1_r, inv_r):
        asum = t0_r[...] + t1_r[...]
        deg = asum[:, 1:2]
        inv = 1.0 / jnp.maximum(deg, 1.0)
        tagg = asum[:, 0:1] * inv
        topo_out = jax.nn.relu((topo_r[...] + tagg) * Wt_r[...] + bt_r[...])
        o1_r[...] = jnp.concatenate([topo_out, jnp.zeros((R, 64), F32)], axis=1)
        inv_r[...] = inv

    return pl.pallas_call(
        body,
        grid=grid,
        in_specs=[
            _rows((1,), R), _rows((DH,), R), _rows((DH,), R),
            _full((1, 64)), _full((1, 64)),
        ],
        out_specs=[_rows((DH,), R), _rows((1,), R)],
        out_shape=[
            jax.ShapeDtypeStruct((N_PAD, DH), F32),
            jax.ShapeDtypeStruct((N_PAD, 1), F32),
        ],
    )(topo, t0, t1, Wt, bt)


def _tc_gcn(x0, x1, a0, a1, inv, W, b, w0, w1):
    R = 512
    grid = (N_PAD // R,)

    def body(x0_r, x1_r, a0_r, a1_r, inv_r, W_r, b_r, o0_r, o1_r):
        x = jnp.concatenate([x0_r[...][:, :w0], x1_r[...][:, :w1]], axis=1)
        a = jnp.concatenate([a0_r[...][:, :w0], a1_r[...][:, :w1]], axis=1) * inv_r[...]
        y = jax.nn.relu(_mm(x + a, W_r[...]) + b_r[...])
        o0_r[...] = y[:, :128]
        o1_r[...] = y[:, 128:]

    return pl.pallas_call(
        body,
        grid=grid,
        in_specs=[
            _rows((DH,), R), _rows((DH,), R),
            _rows((DH,), R), _rows((DH,), R),
            _rows((1,), R),
            _full((w0 + w1, 256)), _full((1, 256)),
        ],
        out_specs=[_rows((DH,), R), _rows((DH,), R)],
        out_shape=[
            jax.ShapeDtypeStruct((N_PAD, DH), F32),
            jax.ShapeDtypeStruct((N_PAD, DH), F32),
        ],
    )(x0, x1, a0, a1, inv, W, b)


def _tc_final(x0, x1, a0, a1, inv, W3, b3, Wf1, bf1, Wf2, bf2, Wf3, bf3):
    R = 400
    grid = (N // R,)

    def body(x0_r, x1_r, a0_r, a1_r, inv_r,
             W3_r, b3_r, Wf1_r, bf1_r, Wf2_r, bf2_r, Wf3_r, bf3_r, o_r):
        x = jnp.concatenate([x0_r[...], x1_r[...]], axis=1)
        a = jnp.concatenate([a0_r[...], a1_r[...]], axis=1) * inv_r[...]
        x3 = jax.nn.relu(_mm(x + a, W3_r[...]) + b3_r[...])
        h = jax.nn.relu(_mm(x3, Wf1_r[...]) + bf1_r[...])
        h = jax.nn.relu(_mm(h, Wf2_r[...]) + bf2_r[...])
        logits = _mm(h, Wf3_r[...]) + bf3_r[...]
        o_r[...] = jax.nn.sigmoid(logits)

    return pl.pallas_call(
        body,
        grid=grid,
        in_specs=[
            _rows((DH,), R), _rows((DH,), R),
            _rows((DH,), R), _rows((DH,), R),
            _rows((1,), R),
            _full((256, 128)), _full((1, 128)),
            _full((128, 128)), _full((1, 128)),
            _full((128, 64)), _full((1, 64)),
            _full((64, 25)), _full((1, 25)),
        ],
        out_specs=[_rows((25,), R)],
        out_shape=[jax.ShapeDtypeStruct((N, 25), F32)],
    )(x0, x1, a0, a1, inv, W3, b3, Wf1, bf1, Wf2, bf2, Wf3, bf3)[0]


def kernel(geom_feat, boundary_pts, topo_feat, face_edge_index,
           Wg1, bg1, Wg2, bg2, Wb1, bb1, Wb2, bb2, Wt, bt,
           W1, b1, W2, b2, W3, b3, Wf1, bf1, Wf2, bf2, Wf3, bf3):
    fe = face_edge_index.astype(jnp.int32)
    pad = jnp.full((E_PAD - E,), N, jnp.int32)
    src = jnp.concatenate([fe[0], pad])
    dst = jnp.concatenate([fe[1], pad])
    src_big = src.reshape(16, E_PAD // 16 // 128, 128)
    dst_big = dst.reshape(16, E_PAD // 16 // 128, 128)
    src_t = src.reshape(32, E_PAD // 32 // 128, 128)
    dst_t = dst.reshape(32, E_PAD // 32 // 128, 128)

    topo128 = jnp.zeros((N_PAD, DH), F32)
    topo128 = topo128.at[:, 1].set(1.0)
    topo128 = topo128.at[:N, 0].set(topo_feat[:, 0])

    t0, t1 = _make_sc_topo()(topo128, src_t, dst_t)

    x0 = _tc_stage_a1(
        geom_feat, boundary_pts,
        Wg1, bg1.reshape(1, -1), Wg2, bg2.reshape(1, -1),
        Wb1, bb1.reshape(1, -1), Wb2, bb2.reshape(1, -1))
    x1, inv = _tc_stage_a2(topo_feat, t0, t1, Wt, bt.reshape(1, -1))

    a0, a1 = _make_sc_agg()(x0, x1, src_big, dst_big)
    y0, y1 = _tc_gcn(x0, x1, a0, a1, inv, W1, b1.reshape(1, -1), 128, 64)

    a0, a1 = _make_sc_agg()(y0, y1, src_big, dst_big)
    z0, z1 = _tc_gcn(y0, y1, a0, a1, inv, W2, b2.reshape(1, -1), 128, 128)

    a0, a1 = _make_sc_agg()(z0, z1, src_big, dst_big)
    out = _tc_final(z0, z1, a0, a1, inv,
                    W3, b3.reshape(1, -1),
                    Wf1[:128], bf1.reshape(1, -1),
                    Wf2, bf2.reshape(1, -1),
                    Wf3, bf3.reshape(1, -1))
    return out
